# baseline (speedup 1.0000x reference)
import functools

import jax
import jax.numpy as jnp
from jax import lax
from jax.experimental import pallas as pl
from jax.experimental.pallas import tpu as pltpu
from jax.experimental.pallas import tpu_sc as plsc


def _capsule_concat_sc(a1r, a2, ft):
    N, M, D = a2.shape
    R = 1 + 2 * M
    NW = 32
    C = 12
    lo_cnt = N // NW
    n_hi = N - lo_cnt * NW
    n_full = lo_cnt // C
    assert lo_cnt % C == 0 and n_hi * (lo_cnt + 1) + (NW - n_hi) * lo_cnt == N

    mesh = plsc.VectorSubcoreMesh(core_axis_name="c", subcore_axis_name="s")

    @functools.partial(
        pl.kernel,
        mesh=mesh,
        out_type=jax.ShapeDtypeStruct((N, R, D), jnp.float32),
        scratch_types=[
            pltpu.VMEM((C, R, D), jnp.float32),
        ],
    )
    def k(a1_hbm, a2_hbm, ft_hbm, out_hbm, stage):
        wid = lax.axis_index("s") * 2 + lax.axis_index("c")
        base_w = wid * lo_cnt + jnp.minimum(wid, n_hi)

        def body(i, carry):
            base = base_w + i * C
            pltpu.sync_copy(a1_hbm.at[pl.ds(base, C)], stage.at[:, pl.ds(0, 1)])
            pltpu.sync_copy(a2_hbm.at[pl.ds(base, C)], stage.at[:, pl.ds(1, M)])
            pltpu.sync_copy(ft_hbm.at[pl.ds(base, C)], stage.at[:, pl.ds(1 + M, M)])
            pltpu.sync_copy(stage, out_hbm.at[pl.ds(base, C)])
            return carry

        lax.fori_loop(0, n_full, body, 0)

        @pl.when(wid < n_hi)
        def _tail():
            base = base_w + n_full * C
            pltpu.sync_copy(a1_hbm.at[pl.ds(base, 1)], stage.at[pl.ds(0, 1), pl.ds(0, 1)])
            pltpu.sync_copy(a2_hbm.at[pl.ds(base, 1)], stage.at[pl.ds(0, 1), pl.ds(1, M)])
            pltpu.sync_copy(ft_hbm.at[pl.ds(base, 1)], stage.at[pl.ds(0, 1), pl.ds(1 + M, M)])
            pltpu.sync_copy(stage.at[pl.ds(0, 1)], out_hbm.at[pl.ds(base, 1)])

    return k(a1r, a2, ft)


@jax.jit
def kernel(a1, a2, ft):
    a1r = a1[:, None, :]
    return _capsule_concat_sc(a1r, a2, ft)


# speedup vs baseline: 1.1808x; 1.1808x over previous
import functools

import jax
import jax.numpy as jnp
from jax import lax
from jax.experimental import pallas as pl
from jax.experimental.pallas import tpu as pltpu
from jax.experimental.pallas import tpu_sc as plsc


def _capsule_concat_sc(a1r, a2, ft):
    N, M, D = a2.shape
    R = 1 + 2 * M
    NW = 32
    C = 8
    NBUF = 3
    lo_cnt = N // NW
    n_hi = N - lo_cnt * NW
    n_full = lo_cnt // C
    assert lo_cnt % C == 0 and n_full % NBUF == 0

    mesh = plsc.VectorSubcoreMesh(core_axis_name="c", subcore_axis_name="s")

    @functools.partial(
        pl.kernel,
        mesh=mesh,
        out_type=jax.ShapeDtypeStruct((N, R, D), jnp.float32),
        scratch_types=(
            [pltpu.VMEM((C, R, D), jnp.float32)] * NBUF
            + [pltpu.SemaphoreType.DMA] * (2 * NBUF)
        ),
    )
    def k(a1_hbm, a2_hbm, ft_hbm, out_hbm, *scratch):
        bufs = scratch[:NBUF]
        isems = scratch[NBUF:2 * NBUF]
        osems = scratch[2 * NBUF:]
        wid = lax.axis_index("s") * 2 + lax.axis_index("c")
        base_w = wid * lo_cnt + jnp.minimum(wid, n_hi)

        def in_copies(b, sem, base):
            return (
                pltpu.make_async_copy(
                    a1_hbm.at[pl.ds(base, C)], bufs[b].at[:, pl.ds(0, 1)], sem),
                pltpu.make_async_copy(
                    a2_hbm.at[pl.ds(base, C)], bufs[b].at[:, pl.ds(1, M)], sem),
                pltpu.make_async_copy(
                    ft_hbm.at[pl.ds(base, C)], bufs[b].at[:, pl.ds(1 + M, M)], sem),
            )

        def out_copy(b, base):
            return pltpu.make_async_copy(bufs[b], out_hbm.at[pl.ds(base, C)], osems[b])

        for b in range(NBUF - 1):
            for c in in_copies(b, isems[b], base_w + b * C):
                c.start()

        def body(j, carry):
            i0 = j * NBUF
            for b in range(NBUF):
                i = i0 + b
                base = base_w + i * C
                pb = (b + NBUF - 1) % NBUF

                @pl.when(i + NBUF - 1 < n_full)
                def _prefetch(pb=pb, i=i, base=base):
                    @pl.when(i >= 1)
                    def _drain():
                        out_copy(pb, base - C).wait()
                    for c in in_copies(pb, isems[pb], base + (NBUF - 1) * C):
                        c.start()

                for c in in_copies(b, isems[b], base):
                    c.wait()
                out_copy(b, base).start()
            return carry

        lax.fori_loop(0, n_full // NBUF, body, 0)

        for b in range(NBUF):
            i = n_full - NBUF + b
            out_copy(i % NBUF, base_w + i * C).wait()

        @pl.when(wid < n_hi)
        def _tail():
            base = base_w + n_full * C
            s = isems[0]
            pltpu.make_async_copy(
                a1_hbm.at[pl.ds(base, 1)], bufs[0].at[pl.ds(0, 1), pl.ds(0, 1)], s
            ).start()
            pltpu.make_async_copy(
                a2_hbm.at[pl.ds(base, 1)], bufs[0].at[pl.ds(0, 1), pl.ds(1, M)], s
            ).start()
            c = pltpu.make_async_copy(
                ft_hbm.at[pl.ds(base, 1)], bufs[0].at[pl.ds(0, 1), pl.ds(1 + M, M)], s)
            c.start()
            pltpu.make_async_copy(
                a1_hbm.at[pl.ds(base, 1)], bufs[0].at[pl.ds(0, 1), pl.ds(0, 1)], s
            ).wait()
            pltpu.make_async_copy(
                a2_hbm.at[pl.ds(base, 1)], bufs[0].at[pl.ds(0, 1), pl.ds(1, M)], s
            ).wait()
            c.wait()
            pltpu.sync_copy(bufs[0].at[pl.ds(0, 1)], out_hbm.at[pl.ds(base, 1)])

    return k(a1r, a2, ft)


@jax.jit
def kernel(a1, a2, ft):
    a1r = a1[:, None, :]
    return _capsule_concat_sc(a1r, a2, ft)


# speedup vs baseline: 1.2628x; 1.0695x over previous
import functools

import jax
import jax.numpy as jnp
from jax.experimental import pallas as pl
from jax.experimental.pallas import tpu as pltpu


def _body(a1_ref, a2_ref, ft_ref, out_ref):
    M = a2_ref.shape[1]
    out_ref[:, 0:1, :] = a1_ref[...][:, None, :]
    out_ref[:, 1:1 + M, :] = a2_ref[...]
    out_ref[:, 1 + M:1 + 2 * M, :] = ft_ref[...]


@jax.jit
def kernel(a1, a2, ft):
    N, M, D = a2.shape
    R = 1 + 2 * M
    Bn = 200
    assert N % Bn == 0
    return pl.pallas_call(
        _body,
        grid=(N // Bn,),
        in_specs=[
            pl.BlockSpec((Bn, D), lambda i: (i, 0)),
            pl.BlockSpec((Bn, M, D), lambda i: (i, 0, 0)),
            pl.BlockSpec((Bn, M, D), lambda i: (i, 0, 0)),
        ],
        out_specs=pl.BlockSpec((Bn, R, D), lambda i: (i, 0, 0)),
        out_shape=jax.ShapeDtypeStruct((N, R, D), jnp.float32),
    )(a1, a2, ft)


# speedup vs baseline: 2.7984x; 2.2160x over previous
import jax
import jax.numpy as jnp
from jax.experimental import pallas as pl
from jax.experimental.pallas import tpu as pltpu


def _body(a1_ref, a2_ref, ft_ref, q_ref):
    Mc = a2_ref.shape[1]
    M = (q_ref.shape[0] - 1) // 2
    k = pl.program_id(1)

    @pl.when(k == 0)
    def _():
        q_ref[0, :, :] = a1_ref[...]

    q_ref[pl.ds(1 + Mc * k, Mc), :, :] = jnp.swapaxes(a2_ref[...], 0, 1)
    q_ref[pl.ds(1 + M + Mc * k, Mc), :, :] = jnp.swapaxes(ft_ref[...], 0, 1)


@jax.jit
def kernel(a1, a2, ft):
    N, M, D = a2.shape
    R = 1 + 2 * M
    Bn = 1000
    Mc = 8
    assert N % Bn == 0 and M % Mc == 0
    q = pl.pallas_call(
        _body,
        grid=(N // Bn, M // Mc),
        in_specs=[
            pl.BlockSpec((Bn, D), lambda i, k: (i, 0)),
            pl.BlockSpec((Bn, Mc, D), lambda i, k: (i, k, 0)),
            pl.BlockSpec((Bn, Mc, D), lambda i, k: (i, k, 0)),
        ],
        out_specs=pl.BlockSpec((R, Bn, D), lambda i, k: (0, i, 0)),
        out_shape=jax.ShapeDtypeStruct((R, N, D), jnp.float32),
        compiler_params=pltpu.CompilerParams(
            dimension_semantics=("parallel", "arbitrary"),
        ),
    )(a1, a2, ft)
    return jnp.transpose(q, (1, 0, 2))
